# baseline (speedup 1.0000x reference)
import functools

import jax
import jax.numpy as jnp
from jax import lax
from jax.experimental import pallas as pl
from jax.experimental.pallas import tpu as pltpu
from jax.experimental.pallas import tpu_sc as plsc

N_ATOMS = 10000
N_EDGES = 320000
D_FEAT = 128
D_EDGE = 16
N_DRUGS = 512
DENSE = 256
B = 4096

NC = 2
NS = 16
EPW = N_EDGES // (NC * NS)
NW = NC * NS
CH = 40
NCHT = N_EDGES // CH
NLOC = NCHT // NW
GRP = 8
ACCW = 160
RPT = N_ATOMS // NS
INV_SQRT_D = 1.0 / (128.0 ** 0.5)

ROW_BLK = 256
ABLK = 2000


def _proj_body(x_ref, wq_ref, wk_ref, wv_ref, we_ref, qcat_ref, kv_ref):
    x = x_ref[...]
    q = jnp.dot(x, wq_ref[...], preferred_element_type=jnp.float32)
    k = jnp.dot(x, wk_ref[...], preferred_element_type=jnp.float32)
    v = jnp.dot(x, wv_ref[...], preferred_element_type=jnp.float32)
    qwe = lax.dot_general(q, we_ref[...], (((1,), (1,)), ((), ())),
                          preferred_element_type=jnp.float32)
    qcat_ref[:, 0:128] = q
    qcat_ref[:, 128:144] = qwe
    kv_ref[:, 0:128] = k
    kv_ref[:, 128:256] = v


def _projections(x, Wq, Wk, Wv, We):
    n = N_ATOMS // ABLK
    return pl.pallas_call(
        _proj_body,
        grid=(n,),
        in_specs=[
            pl.BlockSpec((ABLK, D_FEAT), lambda i: (i, 0)),
            pl.BlockSpec((D_FEAT, D_FEAT), lambda i: (0, 0)),
            pl.BlockSpec((D_FEAT, D_FEAT), lambda i: (0, 0)),
            pl.BlockSpec((D_FEAT, D_FEAT), lambda i: (0, 0)),
            pl.BlockSpec((D_EDGE, D_FEAT), lambda i: (0, 0)),
        ],
        out_specs=[
            pl.BlockSpec((ABLK, 144), lambda i: (i, 0)),
            pl.BlockSpec((ABLK, 256), lambda i: (i, 0)),
        ],
        out_shape=[
            jax.ShapeDtypeStruct((N_ATOMS, 144), jnp.float32),
            jax.ShapeDtypeStruct((N_ATOMS, 256), jnp.float32),
        ],
    )(x, Wq, Wk, Wv, We)


def _edge_sc_body(q_hbm, qwe_hbm, kv_hbm, ea_hbm, src_hbm, dst_hbm, out_hbm,
                  srcv, dstv, dsts, qbuf, qwbuf, kvbuf, eabuf, stage, acc,
                  sem_i, sem_g, sem_s):
    c = lax.axis_index("c")
    s = lax.axis_index("s")
    wid = c * NS + s
    zero16 = jnp.zeros((16,), jnp.float32)
    oh0 = (lax.iota(jnp.int32, 16) == 0).astype(jnp.float32)

    def cbase(g):
        cid = wid + NW * g
        return jnp.where(cid < NCHT, cid, wid) * CH

    def zero_stage(b):
        def zrow(r, _):
            for t in range(ACCW // 16):
                stage[b, r, pl.ds(t * 16, 16)] = zero16
            return _
        lax.fori_loop(0, CH, zrow, 0)

    for b in range(2):
        zero_stage(b)
        for t in range(CH // 16 + 1):
            off = min(t * 16, CH - 16)
            dsts[b, pl.ds(off, 16)] = jnp.zeros((16,), jnp.int32)
    for j in range(RPT // CH):
        pltpu.sync_copy(stage.at[0], acc.at[pl.ds(s * RPT + j * CH, CH)])
    rem = RPT % CH
    if rem:
        pltpu.sync_copy(stage.at[0, pl.ds(0, rem)],
                        acc.at[pl.ds(s * RPT + (RPT // CH) * CH, rem)])
    plsc.subcore_barrier()

    for b in range(2):
        pltpu.async_copy(stage.at[b], acc.at[dsts.at[b]], sem_s, add=True)

    def drain_scatter():
        pltpu.make_async_copy(out_hbm.at[0, pl.ds(0, CH)],
                              stage.at[0], sem_s).wait()

    def issue_idx(g, p):
        base = cbase(g)
        pltpu.async_copy(src_hbm.at[pl.ds(base, CH)], srcv.at[p], sem_i)
        pltpu.async_copy(dst_hbm.at[pl.ds(base, CH)], dstv.at[p], sem_i)
        pltpu.async_copy(ea_hbm.at[pl.ds(base, CH)], eabuf.at[p], sem_i)

    def drain_idx():
        pltpu.make_async_copy(src_hbm.at[pl.ds(0, CH)], srcv.at[0],
                              sem_i).wait()
        pltpu.make_async_copy(dst_hbm.at[pl.ds(0, CH)], dstv.at[0],
                              sem_i).wait()
        pltpu.make_async_copy(ea_hbm.at[pl.ds(0, CH)], eabuf.at[0],
                              sem_i).wait()

    def issue_gathers(p):
        pltpu.async_copy(kv_hbm.at[srcv.at[p]], kvbuf.at[p], sem_g)
        pltpu.async_copy(q_hbm.at[dstv.at[p]], qbuf.at[p], sem_g)
        pltpu.async_copy(qwe_hbm.at[dstv.at[p]], qwbuf.at[p], sem_g)

    def drain_gathers():
        pltpu.make_async_copy(kv_hbm.at[pl.ds(0, CH)], kvbuf.at[0],
                              sem_g).wait()
        pltpu.make_async_copy(q_hbm.at[pl.ds(0, CH)], qbuf.at[0],
                              sem_g).wait()
        pltpu.make_async_copy(qwe_hbm.at[pl.ds(0, CH)], qwbuf.at[0],
                              sem_g).wait()

    b0 = cbase(0)
    pltpu.sync_copy(src_hbm.at[pl.ds(b0, CH)], srcv.at[0])
    pltpu.sync_copy(dst_hbm.at[pl.ds(b0, CH)], dstv.at[0])
    pltpu.sync_copy(ea_hbm.at[pl.ds(b0, CH)], eabuf.at[0])
    issue_idx(1, 1)
    issue_gathers(0)

    def substep(g, b):
        drain_scatter()
        drain_idx()
        issue_gathers(1 - b)
        drain_gathers()

        def group(gg, _2):
            for u in range(GRP):
                e = gg * GRP + u
                ea16 = eabuf[b, e, :]
                dacc = qwbuf[b, e, :] * ea16
                for t in range(4):
                    kw = kvbuf[b, e, pl.ds(32 * t, 32)]
                    ke, ko = plsc.unpack(kw, format=plsc.PackFormat.INTERLEAVED)
                    qw = qbuf[b, e, pl.ds(32 * t, 32)]
                    qe, qo = plsc.unpack(qw, format=plsc.PackFormat.INTERLEAVED)
                    dacc = dacc + qe * ke
                    dacc = dacc + qo * ko
                for kk in (8, 4, 2, 1):
                    perm = jnp.bitwise_xor(lax.iota(jnp.int32, 16), kk)
                    dacc = dacc + dacc.at[perm].get(mode="promise_in_bounds")
                exv = jnp.exp(dacc * INV_SQRT_D)
                for t in range(4):
                    vw = kvbuf[b, e, pl.ds(128 + 32 * t, 32)]
                    ve, vo = plsc.unpack(vw, format=plsc.PackFormat.INTERLEAVED)
                    stage[b, e, pl.ds(32 * t, 16)] = exv * ve
                    stage[b, e, pl.ds(32 * t + 16, 16)] = exv * vo
                stage[b, e, pl.ds(128, 16)] = exv * ea16
                stage[b, e, pl.ds(144, 16)] = exv * oh0
            return _2
        lax.fori_loop(0, CH // GRP, group, 0)

        @pl.when(wid + NW * g >= NCHT)
        def _dummy():
            zero_stage(b)

        dsts[b, pl.ds(0, 16)] = dstv[b, pl.ds(0, 16)]
        dsts[b, pl.ds(16, 16)] = dstv[b, pl.ds(16, 16)]
        pltpu.async_copy(stage.at[b], acc.at[dsts.at[b]], sem_s, add=True)
        issue_idx(g + 2, b)

    def loop2(j, _):
        substep(2 * j, 0)
        substep(2 * j + 1, 1)
        return _
    lax.fori_loop(0, NLOC // 2, loop2, 0)

    drain_idx()
    drain_gathers()
    drain_scatter()
    drain_scatter()

    plsc.subcore_barrier()
    pltpu.sync_copy(acc.at[pl.ds(s * RPT, RPT)],
                    out_hbm.at[c, pl.ds(s * RPT, RPT)])


def _edge_phase(q_bf, qwe, kv_bf, edge_attr, src, dst):
    mesh = plsc.VectorSubcoreMesh(core_axis_name="c", subcore_axis_name="s")
    f = pl.kernel(
        _edge_sc_body,
        mesh=mesh,
        out_type=jax.ShapeDtypeStruct((NC, N_ATOMS, ACCW), jnp.float32),
        scratch_types=[
            pltpu.VMEM((2, CH), jnp.int32),
            pltpu.VMEM((2, CH), jnp.int32),
            pltpu.VMEM((2, CH), jnp.int32),
            pltpu.VMEM((2, CH, 128), jnp.bfloat16),
            pltpu.VMEM((2, CH, D_EDGE), jnp.float32),
            pltpu.VMEM((2, CH, 256), jnp.bfloat16),
            pltpu.VMEM((2, CH, D_EDGE), jnp.float32),
            pltpu.VMEM((2, CH, ACCW), jnp.float32),
            pltpu.VMEM_SHARED((N_ATOMS, ACCW), jnp.float32),
            pltpu.SemaphoreType.DMA,
            pltpu.SemaphoreType.DMA,
            pltpu.SemaphoreType.DMA,
        ],
        compiler_params=pltpu.CompilerParams(use_tc_tiling_on_sc=False,
                                             needs_layout_passes=False),
    )
    return f(q_bf, qwe, kv_bf, edge_attr, src, dst)


def _pool_body(accv_ref, accr_ref, batch_ref, we_ref, wo_ref, psum_ref,
               cnt_ref):
    i = pl.program_id(0)
    msgv = accv_ref[0] + accv_ref[1]
    accr = accr_ref[0] + accr_ref[1]
    msg = msgv + jnp.dot(accr[:, 0:16], we_ref[...],
                         preferred_element_type=jnp.float32)
    den = accr[:, 16:17]
    h = msg / (den + 1e-16)
    h = jnp.maximum(jnp.dot(h, wo_ref[...], preferred_element_type=jnp.float32),
                    0.0)
    b = batch_ref[0, 0, :]
    oh = (lax.broadcasted_iota(jnp.int32, (N_DRUGS, ABLK), 0) ==
          b[None, :]).astype(jnp.float32)
    ps = jnp.dot(oh, h, preferred_element_type=jnp.float32)
    cs = jnp.dot(oh, jnp.ones((ABLK, 128), jnp.float32),
                 preferred_element_type=jnp.float32)

    @pl.when(i == 0)
    def _init():
        psum_ref[...] = jnp.zeros_like(psum_ref)
        cnt_ref[...] = jnp.zeros_like(cnt_ref)

    psum_ref[...] += ps
    cnt_ref[...] += cs


def _pool(accv, accr, batch, We, Wo):
    n = N_ATOMS // ABLK
    batch3 = batch.astype(jnp.int32).reshape(n, 1, ABLK)
    return pl.pallas_call(
        _pool_body,
        grid=(n,),
        in_specs=[
            pl.BlockSpec((NC, ABLK, 128), lambda i: (0, i, 0)),
            pl.BlockSpec((NC, ABLK, 32), lambda i: (0, i, 0)),
            pl.BlockSpec((1, 1, ABLK), lambda i: (i, 0, 0)),
            pl.BlockSpec((D_EDGE, D_FEAT), lambda i: (0, 0)),
            pl.BlockSpec((D_FEAT, D_FEAT), lambda i: (0, 0)),
        ],
        out_specs=[
            pl.BlockSpec((N_DRUGS, D_FEAT), lambda i: (0, 0)),
            pl.BlockSpec((N_DRUGS, D_FEAT), lambda i: (0, 0)),
        ],
        out_shape=[
            jax.ShapeDtypeStruct((N_DRUGS, D_FEAT), jnp.float32),
            jax.ShapeDtypeStruct((N_DRUGS, D_FEAT), jnp.float32),
        ],
    )(accv, accr, batch3, We, Wo)


def _head_body(psum_ref, cnt_ref, w1_ref, b1_ref, w2_ref, b2_ref, out_ref):
    pooled = psum_ref[...] / (cnt_ref[...] + 1e-16)
    hid = jnp.maximum(
        jnp.dot(pooled, w1_ref[...], preferred_element_type=jnp.float32)
        + b1_ref[...], 0.0)
    out_ref[...] = (jnp.dot(hid, w2_ref[...],
                            preferred_element_type=jnp.float32) + b2_ref[...])


def _head(psum, cnt, W1, b1, W2, b2):
    return pl.pallas_call(
        _head_body,
        out_shape=jax.ShapeDtypeStruct((N_DRUGS, D_FEAT), jnp.float32),
    )(psum, cnt, W1.reshape(1, D_FEAT, DENSE)[0], b1.reshape(1, DENSE),
      W2.reshape(1, DENSE, D_FEAT)[0], b2.reshape(1, D_FEAT))


def _assemble_body(d1_ref, d2_ref, feat_ref, fp1_ref, fp2_ref, dti1_ref,
                   dti2_ref, cell_ref, out_ref):
    pid = pl.program_id(0)
    d1 = d1_ref[pl.ds(pid * ROW_BLK, ROW_BLK)]
    d2 = d2_ref[pl.ds(pid * ROW_BLK, ROW_BLK)]
    drugs = lax.broadcasted_iota(jnp.int32, (ROW_BLK, N_DRUGS), 1)
    oh1 = (d1[:, None] == drugs).astype(jnp.float32)
    oh2 = (d2[:, None] == drugs).astype(jnp.float32)
    table = feat_ref[...]
    out_ref[:, 0:128] = jnp.dot(oh1, table, preferred_element_type=jnp.float32)
    out_ref[:, 128:256] = jnp.dot(oh2, table,
                                  preferred_element_type=jnp.float32)
    out_ref[:, 256:2304] = fp1_ref[...]
    out_ref[:, 2304:4352] = fp2_ref[...]
    out_ref[:, 4352:4608] = dti1_ref[...]
    out_ref[:, 4608:4864] = dti2_ref[...]
    out_ref[:, 4864:5632] = cell_ref[...]


def _assemble(d1, d2, all_drug_feat, fp1, fp2, dti1, dti2, cell):
    nblk = B // ROW_BLK
    row_spec = lambda w: pl.BlockSpec((ROW_BLK, w), lambda i: (i, 0))
    return pl.pallas_call(
        _assemble_body,
        grid=(nblk,),
        in_specs=[
            pl.BlockSpec((B,), lambda i: (0,)),
            pl.BlockSpec((B,), lambda i: (0,)),
            pl.BlockSpec((N_DRUGS, D_FEAT), lambda i: (0, 0)),
            row_spec(2048), row_spec(2048), row_spec(256), row_spec(256),
            row_spec(768),
        ],
        out_specs=pl.BlockSpec((ROW_BLK, 5632), lambda i: (i, 0)),
        out_shape=jax.ShapeDtypeStruct((B, 5632), jnp.float32),
    )(d1, d2, all_drug_feat, fp1, fp2, dti1, dti2, cell)


def kernel(drug1_idx, drug2_idx, drug1_fp, drug2_fp, drug1_dti, drug2_dti,
           cell_feat, x, edge_attr, edge_index, batch, Wq, Wk, Wv, We, Wo,
           W1, b1, W2, b2):
    d1 = jnp.ravel(drug1_idx).astype(jnp.int32)
    d2 = jnp.ravel(drug2_idx).astype(jnp.int32)
    src = edge_index[0].astype(jnp.int32)
    dst = edge_index[1].astype(jnp.int32)

    qcat0, kv = _projections(x, Wq, Wk, Wv, We)
    q_bf = qcat0[:, :128].astype(jnp.bfloat16)
    qwe = qcat0[:, 128:144]
    kv_bf = kv.astype(jnp.bfloat16)
    acc = _edge_phase(q_bf, qwe, kv_bf, edge_attr, src, dst)
    accv = (acc[..., :128].reshape(NC, N_ATOMS, 4, 2, 16)
            .transpose(0, 1, 2, 4, 3).reshape(NC, N_ATOMS, 128))
    accr = acc[..., 128:]
    psum, cnt = _pool(accv, accr, batch, We, Wo)
    all_drug_feat = _head(psum, cnt, W1, b1, W2, b2)
    return _assemble(d1, d2, all_drug_feat, drug1_fp, drug2_fp, drug1_dti,
                     drug2_dti, cell_feat)


# speedup vs baseline: 1.0008x; 1.0008x over previous
import functools

import jax
import jax.numpy as jnp
from jax import lax
from jax.experimental import pallas as pl
from jax.experimental.pallas import tpu as pltpu
from jax.experimental.pallas import tpu_sc as plsc

N_ATOMS = 10000
N_EDGES = 320000
D_FEAT = 128
D_EDGE = 16
N_DRUGS = 512
DENSE = 256
B = 4096

NC = 2
NS = 16
EPW = N_EDGES // (NC * NS)
NW = NC * NS
CH = 40
NCHT = N_EDGES // CH
NLOC = NCHT // NW
GRP = 4
ACCW = 160
RPT = N_ATOMS // NS
INV_SQRT_D = 1.0 / (128.0 ** 0.5)

ROW_BLK = 256
ABLK = 2000


def _proj_body(x_ref, wq_ref, wk_ref, wv_ref, we_ref, qcat_ref, kv_ref):
    x = x_ref[...]
    q = jnp.dot(x, wq_ref[...], preferred_element_type=jnp.float32)
    k = jnp.dot(x, wk_ref[...], preferred_element_type=jnp.float32)
    v = jnp.dot(x, wv_ref[...], preferred_element_type=jnp.float32)
    qwe = lax.dot_general(q, we_ref[...], (((1,), (1,)), ((), ())),
                          preferred_element_type=jnp.float32)
    qcat_ref[:, 0:128] = q
    qcat_ref[:, 128:144] = qwe
    kv_ref[:, 0:128] = k
    kv_ref[:, 128:256] = v


def _projections(x, Wq, Wk, Wv, We):
    n = N_ATOMS // ABLK
    return pl.pallas_call(
        _proj_body,
        grid=(n,),
        in_specs=[
            pl.BlockSpec((ABLK, D_FEAT), lambda i: (i, 0)),
            pl.BlockSpec((D_FEAT, D_FEAT), lambda i: (0, 0)),
            pl.BlockSpec((D_FEAT, D_FEAT), lambda i: (0, 0)),
            pl.BlockSpec((D_FEAT, D_FEAT), lambda i: (0, 0)),
            pl.BlockSpec((D_EDGE, D_FEAT), lambda i: (0, 0)),
        ],
        out_specs=[
            pl.BlockSpec((ABLK, 144), lambda i: (i, 0)),
            pl.BlockSpec((ABLK, 256), lambda i: (i, 0)),
        ],
        out_shape=[
            jax.ShapeDtypeStruct((N_ATOMS, 144), jnp.float32),
            jax.ShapeDtypeStruct((N_ATOMS, 256), jnp.float32),
        ],
    )(x, Wq, Wk, Wv, We)


def _edge_sc_body(q_hbm, qwe_hbm, kv_hbm, ea_hbm, src_hbm, dst_hbm, out_hbm,
                  srcv, dstv, dsts, qbuf, qwbuf, kvbuf, eabuf, stage, acc,
                  sem_i, sem_g, sem_s):
    c = lax.axis_index("c")
    s = lax.axis_index("s")
    wid = c * NS + s
    zero16 = jnp.zeros((16,), jnp.float32)
    oh0 = (lax.iota(jnp.int32, 16) == 0).astype(jnp.float32)

    def cbase(g):
        cid = wid + NW * g
        return jnp.where(cid < NCHT, cid, wid) * CH

    def zero_stage(b):
        def zrow(r, _):
            for t in range(ACCW // 16):
                stage[b, r, pl.ds(t * 16, 16)] = zero16
            return _
        lax.fori_loop(0, CH, zrow, 0)

    for b in range(2):
        zero_stage(b)
        for t in range(CH // 16 + 1):
            off = min(t * 16, CH - 16)
            dsts[b, pl.ds(off, 16)] = jnp.zeros((16,), jnp.int32)
    for j in range(RPT // CH):
        pltpu.sync_copy(stage.at[0], acc.at[pl.ds(s * RPT + j * CH, CH)])
    rem = RPT % CH
    if rem:
        pltpu.sync_copy(stage.at[0, pl.ds(0, rem)],
                        acc.at[pl.ds(s * RPT + (RPT // CH) * CH, rem)])
    plsc.subcore_barrier()

    for b in range(2):
        pltpu.async_copy(stage.at[b], acc.at[dsts.at[b]], sem_s, add=True)

    def drain_scatter():
        pltpu.make_async_copy(out_hbm.at[0, pl.ds(0, CH)],
                              stage.at[0], sem_s).wait()

    def issue_idx(g, p):
        base = cbase(g)
        pltpu.async_copy(src_hbm.at[pl.ds(base, CH)], srcv.at[p], sem_i)
        pltpu.async_copy(dst_hbm.at[pl.ds(base, CH)], dstv.at[p], sem_i)
        pltpu.async_copy(ea_hbm.at[pl.ds(base, CH)], eabuf.at[p], sem_i)

    def drain_idx():
        pltpu.make_async_copy(src_hbm.at[pl.ds(0, CH)], srcv.at[0],
                              sem_i).wait()
        pltpu.make_async_copy(dst_hbm.at[pl.ds(0, CH)], dstv.at[0],
                              sem_i).wait()
        pltpu.make_async_copy(ea_hbm.at[pl.ds(0, CH)], eabuf.at[0],
                              sem_i).wait()

    def issue_gathers(p):
        pltpu.async_copy(kv_hbm.at[srcv.at[p]], kvbuf.at[p], sem_g)
        pltpu.async_copy(q_hbm.at[dstv.at[p]], qbuf.at[p], sem_g)
        pltpu.async_copy(qwe_hbm.at[dstv.at[p]], qwbuf.at[p], sem_g)

    def drain_gathers():
        pltpu.make_async_copy(kv_hbm.at[pl.ds(0, CH)], kvbuf.at[0],
                              sem_g).wait()
        pltpu.make_async_copy(q_hbm.at[pl.ds(0, CH)], qbuf.at[0],
                              sem_g).wait()
        pltpu.make_async_copy(qwe_hbm.at[pl.ds(0, CH)], qwbuf.at[0],
                              sem_g).wait()

    b0 = cbase(0)
    pltpu.sync_copy(src_hbm.at[pl.ds(b0, CH)], srcv.at[0])
    pltpu.sync_copy(dst_hbm.at[pl.ds(b0, CH)], dstv.at[0])
    pltpu.sync_copy(ea_hbm.at[pl.ds(b0, CH)], eabuf.at[0])
    issue_idx(1, 1)
    issue_gathers(0)

    def substep(g, b):
        drain_scatter()
        drain_idx()
        issue_gathers(1 - b)
        drain_gathers()

        def group(gg, _2):
            for u in range(GRP):
                e = gg * GRP + u
                ea16 = eabuf[b, e, :]
                dacc = qwbuf[b, e, :] * ea16
                for t in range(4):
                    kw = kvbuf[b, e, pl.ds(32 * t, 32)]
                    ke, ko = plsc.unpack(kw, format=plsc.PackFormat.INTERLEAVED)
                    qw = qbuf[b, e, pl.ds(32 * t, 32)]
                    qe, qo = plsc.unpack(qw, format=plsc.PackFormat.INTERLEAVED)
                    dacc = dacc + qe * ke
                    dacc = dacc + qo * ko
                for kk in (8, 4, 2, 1):
                    perm = jnp.bitwise_xor(lax.iota(jnp.int32, 16), kk)
                    dacc = dacc + dacc.at[perm].get(mode="promise_in_bounds")
                exv = jnp.exp(dacc * INV_SQRT_D)
                for t in range(4):
                    vw = kvbuf[b, e, pl.ds(128 + 32 * t, 32)]
                    ve, vo = plsc.unpack(vw, format=plsc.PackFormat.INTERLEAVED)
                    stage[b, e, pl.ds(32 * t, 16)] = exv * ve
                    stage[b, e, pl.ds(32 * t + 16, 16)] = exv * vo
                stage[b, e, pl.ds(128, 16)] = exv * ea16
                stage[b, e, pl.ds(144, 16)] = exv * oh0
            return _2
        lax.fori_loop(0, CH // GRP, group, 0)

        @pl.when(wid + NW * g >= NCHT)
        def _dummy():
            zero_stage(b)

        dsts[b, pl.ds(0, 16)] = dstv[b, pl.ds(0, 16)]
        dsts[b, pl.ds(16, 16)] = dstv[b, pl.ds(16, 16)]
        pltpu.async_copy(stage.at[b], acc.at[dsts.at[b]], sem_s, add=True)
        issue_idx(g + 2, b)

    def loop2(j, _):
        substep(2 * j, 0)
        substep(2 * j + 1, 1)
        return _
    lax.fori_loop(0, NLOC // 2, loop2, 0)

    drain_idx()
    drain_gathers()
    drain_scatter()
    drain_scatter()

    plsc.subcore_barrier()
    pltpu.sync_copy(acc.at[pl.ds(s * RPT, RPT)],
                    out_hbm.at[c, pl.ds(s * RPT, RPT)])


def _edge_phase(q_bf, qwe, kv_bf, edge_attr, src, dst):
    mesh = plsc.VectorSubcoreMesh(core_axis_name="c", subcore_axis_name="s")
    f = pl.kernel(
        _edge_sc_body,
        mesh=mesh,
        out_type=jax.ShapeDtypeStruct((NC, N_ATOMS, ACCW), jnp.float32),
        scratch_types=[
            pltpu.VMEM((2, CH), jnp.int32),
            pltpu.VMEM((2, CH), jnp.int32),
            pltpu.VMEM((2, CH), jnp.int32),
            pltpu.VMEM((2, CH, 128), jnp.bfloat16),
            pltpu.VMEM((2, CH, D_EDGE), jnp.float32),
            pltpu.VMEM((2, CH, 256), jnp.bfloat16),
            pltpu.VMEM((2, CH, D_EDGE), jnp.float32),
            pltpu.VMEM((2, CH, ACCW), jnp.float32),
            pltpu.VMEM_SHARED((N_ATOMS, ACCW), jnp.float32),
            pltpu.SemaphoreType.DMA,
            pltpu.SemaphoreType.DMA,
            pltpu.SemaphoreType.DMA,
        ],
        compiler_params=pltpu.CompilerParams(use_tc_tiling_on_sc=False,
                                             needs_layout_passes=False),
    )
    return f(q_bf, qwe, kv_bf, edge_attr, src, dst)


def _pool_body(accv_ref, accr_ref, batch_ref, we_ref, wo_ref, psum_ref,
               cnt_ref):
    i = pl.program_id(0)
    msgv = accv_ref[0] + accv_ref[1]
    accr = accr_ref[0] + accr_ref[1]
    msg = msgv + jnp.dot(accr[:, 0:16], we_ref[...],
                         preferred_element_type=jnp.float32)
    den = accr[:, 16:17]
    h = msg / (den + 1e-16)
    h = jnp.maximum(jnp.dot(h, wo_ref[...], preferred_element_type=jnp.float32),
                    0.0)
    b = batch_ref[0, 0, :]
    oh = (lax.broadcasted_iota(jnp.int32, (N_DRUGS, ABLK), 0) ==
          b[None, :]).astype(jnp.float32)
    ps = jnp.dot(oh, h, preferred_element_type=jnp.float32)
    cs = jnp.dot(oh, jnp.ones((ABLK, 128), jnp.float32),
                 preferred_element_type=jnp.float32)

    @pl.when(i == 0)
    def _init():
        psum_ref[...] = jnp.zeros_like(psum_ref)
        cnt_ref[...] = jnp.zeros_like(cnt_ref)

    psum_ref[...] += ps
    cnt_ref[...] += cs


def _pool(accv, accr, batch, We, Wo):
    n = N_ATOMS // ABLK
    batch3 = batch.astype(jnp.int32).reshape(n, 1, ABLK)
    return pl.pallas_call(
        _pool_body,
        grid=(n,),
        in_specs=[
            pl.BlockSpec((NC, ABLK, 128), lambda i: (0, i, 0)),
            pl.BlockSpec((NC, ABLK, 32), lambda i: (0, i, 0)),
            pl.BlockSpec((1, 1, ABLK), lambda i: (i, 0, 0)),
            pl.BlockSpec((D_EDGE, D_FEAT), lambda i: (0, 0)),
            pl.BlockSpec((D_FEAT, D_FEAT), lambda i: (0, 0)),
        ],
        out_specs=[
            pl.BlockSpec((N_DRUGS, D_FEAT), lambda i: (0, 0)),
            pl.BlockSpec((N_DRUGS, D_FEAT), lambda i: (0, 0)),
        ],
        out_shape=[
            jax.ShapeDtypeStruct((N_DRUGS, D_FEAT), jnp.float32),
            jax.ShapeDtypeStruct((N_DRUGS, D_FEAT), jnp.float32),
        ],
    )(accv, accr, batch3, We, Wo)


def _head_body(psum_ref, cnt_ref, w1_ref, b1_ref, w2_ref, b2_ref, out_ref):
    pooled = psum_ref[...] / (cnt_ref[...] + 1e-16)
    hid = jnp.maximum(
        jnp.dot(pooled, w1_ref[...], preferred_element_type=jnp.float32)
        + b1_ref[...], 0.0)
    out_ref[...] = (jnp.dot(hid, w2_ref[...],
                            preferred_element_type=jnp.float32) + b2_ref[...])


def _head(psum, cnt, W1, b1, W2, b2):
    return pl.pallas_call(
        _head_body,
        out_shape=jax.ShapeDtypeStruct((N_DRUGS, D_FEAT), jnp.float32),
    )(psum, cnt, W1.reshape(1, D_FEAT, DENSE)[0], b1.reshape(1, DENSE),
      W2.reshape(1, DENSE, D_FEAT)[0], b2.reshape(1, D_FEAT))


def _assemble_body(d1_ref, d2_ref, feat_ref, fp1_ref, fp2_ref, dti1_ref,
                   dti2_ref, cell_ref, out_ref):
    pid = pl.program_id(0)
    d1 = d1_ref[pl.ds(pid * ROW_BLK, ROW_BLK)]
    d2 = d2_ref[pl.ds(pid * ROW_BLK, ROW_BLK)]
    drugs = lax.broadcasted_iota(jnp.int32, (ROW_BLK, N_DRUGS), 1)
    oh1 = (d1[:, None] == drugs).astype(jnp.float32)
    oh2 = (d2[:, None] == drugs).astype(jnp.float32)
    table = feat_ref[...]
    out_ref[:, 0:128] = jnp.dot(oh1, table, preferred_element_type=jnp.float32)
    out_ref[:, 128:256] = jnp.dot(oh2, table,
                                  preferred_element_type=jnp.float32)
    out_ref[:, 256:2304] = fp1_ref[...]
    out_ref[:, 2304:4352] = fp2_ref[...]
    out_ref[:, 4352:4608] = dti1_ref[...]
    out_ref[:, 4608:4864] = dti2_ref[...]
    out_ref[:, 4864:5632] = cell_ref[...]


def _assemble(d1, d2, all_drug_feat, fp1, fp2, dti1, dti2, cell):
    nblk = B // ROW_BLK
    row_spec = lambda w: pl.BlockSpec((ROW_BLK, w), lambda i: (i, 0))
    return pl.pallas_call(
        _assemble_body,
        grid=(nblk,),
        in_specs=[
            pl.BlockSpec((B,), lambda i: (0,)),
            pl.BlockSpec((B,), lambda i: (0,)),
            pl.BlockSpec((N_DRUGS, D_FEAT), lambda i: (0, 0)),
            row_spec(2048), row_spec(2048), row_spec(256), row_spec(256),
            row_spec(768),
        ],
        out_specs=pl.BlockSpec((ROW_BLK, 5632), lambda i: (i, 0)),
        out_shape=jax.ShapeDtypeStruct((B, 5632), jnp.float32),
    )(d1, d2, all_drug_feat, fp1, fp2, dti1, dti2, cell)


def kernel(drug1_idx, drug2_idx, drug1_fp, drug2_fp, drug1_dti, drug2_dti,
           cell_feat, x, edge_attr, edge_index, batch, Wq, Wk, Wv, We, Wo,
           W1, b1, W2, b2):
    d1 = jnp.ravel(drug1_idx).astype(jnp.int32)
    d2 = jnp.ravel(drug2_idx).astype(jnp.int32)
    src = edge_index[0].astype(jnp.int32)
    dst = edge_index[1].astype(jnp.int32)

    qcat0, kv = _projections(x, Wq, Wk, Wv, We)
    q_bf = qcat0[:, :128].astype(jnp.bfloat16)
    qwe = qcat0[:, 128:144]
    kv_bf = kv.astype(jnp.bfloat16)
    acc = _edge_phase(q_bf, qwe, kv_bf, edge_attr, src, dst)
    accv = (acc[..., :128].reshape(NC, N_ATOMS, 4, 2, 16)
            .transpose(0, 1, 2, 4, 3).reshape(NC, N_ATOMS, 128))
    accr = acc[..., 128:]
    psum, cnt = _pool(accv, accr, batch, We, Wo)
    all_drug_feat = _head(psum, cnt, W1, b1, W2, b2)
    return _assemble(d1, d2, all_drug_feat, drug1_fp, drug2_fp, drug1_dti,
                     drug2_dti, cell_feat)
